# baseline (speedup 1.0000x reference)
import jax
import jax.numpy as jnp
from jax import lax
from jax.experimental import pallas as pl
from jax.experimental.pallas import tpu as pltpu
from jax.experimental.pallas import tpu_sc as plsc

NUM_CLASSES = 1000
DIM = 64
BATCH = 16384
LANES = 16

_info = plsc.get_sparse_core_info()
NUM_CORES = _info.num_cores
NUM_SUBCORES = _info.num_subcores
NW = NUM_CORES * NUM_SUBCORES
B_PER_W = BATCH // NW
CHUNK = 128
NCHUNK = B_PER_W // CHUNK


def _freq_bias_body(sbj_hbm, obj_hbm, table_hbm, out_hbm,
                    sbj_v, obj_v, idx_v, rows_v, sem):
    wid = lax.axis_index("s") * NUM_CORES + lax.axis_index("c")
    base = wid * B_PER_W
    pltpu.sync_copy(sbj_hbm.at[pl.ds(base, B_PER_W)], sbj_v)
    pltpu.sync_copy(obj_hbm.at[pl.ds(base, B_PER_W)], obj_v)
    for j in range(NCHUNK):
        for i in range(CHUNK // LANES):
            off = j * CHUNK + i * LANES
            s = sbj_v[pl.ds(off, LANES)]
            o = obj_v[pl.ds(off, LANES)]
            idx_v[j, pl.ds(i * LANES, LANES)] = s * NUM_CLASSES + o
    copies = [
        pltpu.async_copy(table_hbm.at[idx_v.at[j]],
                         rows_v.at[pl.ds(j * CHUNK, CHUNK)], sem)
        for j in range(NCHUNK)
    ]
    for c in copies:
        c.wait()
    pltpu.sync_copy(rows_v, out_hbm.at[pl.ds(base, B_PER_W)])


def kernel(sbj_labels, obj_labels, node_baseline):
    mesh = plsc.VectorSubcoreMesh(core_axis_name="c", subcore_axis_name="s")
    k = pl.kernel(
        _freq_bias_body,
        mesh=mesh,
        compiler_params=pltpu.CompilerParams(use_tc_tiling_on_sc=False),
        out_type=jax.ShapeDtypeStruct((BATCH, DIM), jnp.float32),
        scratch_types=[
            pltpu.VMEM((B_PER_W,), jnp.int32),
            pltpu.VMEM((B_PER_W,), jnp.int32),
            pltpu.VMEM((NCHUNK, CHUNK), jnp.int32),
            pltpu.VMEM((B_PER_W, DIM), jnp.float32),
            pltpu.SemaphoreType.DMA,
        ],
    )
    return k(sbj_labels.astype(jnp.int32), obj_labels.astype(jnp.int32),
             node_baseline)


# speedup vs baseline: 2.4629x; 2.4629x over previous
import jax
import jax.numpy as jnp
from jax import lax
from jax.experimental import pallas as pl
from jax.experimental.pallas import tpu as pltpu
from jax.experimental.pallas import tpu_sc as plsc

NUM_CLASSES = 1000
DIM = 64
BATCH = 16384
LANES = 16
SUBROWS = 8

_info = plsc.get_sparse_core_info()
NUM_CORES = _info.num_cores
NUM_SUBCORES = _info.num_subcores
NW = NUM_CORES * NUM_SUBCORES
B_PER_W = BATCH // NW
NTILE = 125000


def _freq_bias_body(sbj_hbm, obj_hbm, table_hbm, out_hbm,
                    sbj_v, obj_v, outb_v, sem):
    wid = lax.axis_index("s") * NUM_CORES + lax.axis_index("c")
    base = wid * B_PER_W
    pltpu.sync_copy(sbj_hbm.at[pl.ds(base, B_PER_W)], sbj_v)
    pltpu.sync_copy(obj_hbm.at[pl.ds(base, B_PER_W)], obj_v)

    for g in range(B_PER_W // LANES):
        s = sbj_v[pl.ds(g * LANES, LANES)]
        o = obj_v[pl.ds(g * LANES, LANES)]
        f = s * NUM_CLASSES + o
        t_vec = lax.shift_right_logical(f, 3)
        r_vec = lax.bitwise_and(f, 7)
        for l in range(LANES):
            e = g * LANES + l
            pltpu.async_copy(
                table_hbm.at[t_vec[l], r_vec[l]],
                outb_v.at[e // SUBROWS, e % SUBROWS],
                sem)

    def drain_body(i, _):
        pltpu.make_async_copy(table_hbm.at[0, 0], outb_v.at[0, 0], sem).wait()
        return _

    lax.fori_loop(0, B_PER_W, drain_body, None)

    pltpu.sync_copy(outb_v,
                    out_hbm.at[pl.ds(wid * (B_PER_W // SUBROWS),
                                     B_PER_W // SUBROWS)])


def kernel(sbj_labels, obj_labels, node_baseline):
    mesh = plsc.VectorSubcoreMesh(core_axis_name="c", subcore_axis_name="s")
    k = pl.kernel(
        _freq_bias_body,
        mesh=mesh,
        compiler_params=pltpu.CompilerParams(use_tc_tiling_on_sc=True),
        out_type=jax.ShapeDtypeStruct((BATCH // SUBROWS, SUBROWS, DIM),
                                      jnp.float32),
        scratch_types=[
            pltpu.VMEM((B_PER_W,), jnp.int32),
            pltpu.VMEM((B_PER_W,), jnp.int32),
            pltpu.VMEM((B_PER_W // SUBROWS, SUBROWS, DIM), jnp.float32),
            pltpu.SemaphoreType.DMA,
        ],
    )
    table3 = node_baseline.reshape(NTILE, SUBROWS, DIM)
    out3 = k(sbj_labels.astype(jnp.int32), obj_labels.astype(jnp.int32),
             table3)
    return out3.reshape(BATCH, DIM)
